# baseline (speedup 1.0000x reference)
import functools
import jax
import jax.numpy as jnp
from jax import lax
from jax.experimental import pallas as pl
from jax.experimental.pallas import tpu as pltpu
from jax.experimental.pallas import tpu_sc as plsc

N = 10000
E = 320000
D = 128
R = 32

NC = 2
NS = 16
NW = NC * NS

IB = 128
KROWS = 2
CH = IB * KROWS
TOTAL_CHUNKS = E // CH
GITERS = -(-TOTAL_CHUNKS // NW)
SITERS = -(-(TOTAL_CHUNKS // NC) // NS)
ZCH = 80
ZCHUNKS = N // ZCH
ZITERS = -(-ZCHUNKS // NS)

EBLK = 2000
NSLICES = 10


def _silu(v):
    return v * jax.nn.sigmoid(v)


GROWS = 80
GLAST = (E // IB) - (NW - 1) * GROWS


def _sc_gather_body(xi_hbm, xj_hbm, dst_hbm, src_hbm, g_hbm,
                    idx_d, idx_s, ra0, rb0, ra1, rb1,
                    sa0, sb0, sa1, sb1, so0, so1):
    wid = lax.axis_index("s") * NC + lax.axis_index("c")
    cnt = jnp.where(wid < NW - 1, GROWS, GLAST)

    @pl.when(wid < NW - 1)
    def _():
        pltpu.sync_copy(dst_hbm.at[pl.ds(wid * GROWS, GROWS)], idx_d)
        pltpu.sync_copy(src_hbm.at[pl.ds(wid * GROWS, GROWS)], idx_s)

    @pl.when(wid == NW - 1)
    def _():
        pltpu.sync_copy(dst_hbm.at[pl.ds((NW - 1) * GROWS, GLAST)],
                        idx_d.at[pl.ds(0, GLAST)])
        pltpu.sync_copy(src_hbm.at[pl.ds((NW - 1) * GROWS, GLAST)],
                        idx_s.at[pl.ds(0, GLAST)])

    def cid_of(k):
        return wid * GROWS + k

    def issue(k, ra, rb, sa, sb):
        pltpu.async_copy(xi_hbm.at[idx_d.at[k]], ra, sa)
        pltpu.async_copy(xj_hbm.at[idx_s.at[k]], rb, sb)

    def drain_gather(ra, rb, sa, sb):
        pltpu.make_async_copy(xi_hbm.at[idx_d.at[0]], ra, sa).wait()
        pltpu.make_async_copy(xj_hbm.at[idx_s.at[0]], rb, sb).wait()

    def drain_store(ra, so):
        pltpu.make_async_copy(ra, g_hbm.at[pl.ds(0, IB)], so).wait()

    issue(0, ra0, rb0, sa0, sb0)

    bufs = ((ra0, rb0, sa0, sb0, so0), (ra1, rb1, sa1, sb1, so1))

    def pair(k2, carry):
        for p in (0, 1):
            k = 2 * k2 + p
            ra, rb, sa, sb, so = bufs[p]
            nra, nrb, nsa, nsb, nso = bufs[1 - p]

            @pl.when((k + 1 < cnt) & (k >= 1))
            def _():
                drain_store(nra, nso)

            @pl.when(k + 1 < cnt)
            def _():
                issue(k + 1, nra, nrb, nsa, nsb)

            @pl.when(k < cnt)
            def _():
                drain_gather(ra, rb, sa, sb)

                def add_row(r, c2):
                    for j in range(D // 16):
                        sl = pl.ds(j * 16, 16)
                        ra[r, sl] = ra[r, sl] + rb[r, sl]
                    return c2

                lax.fori_loop(0, IB, add_row, 0)
                pltpu.async_copy(ra, g_hbm.at[pl.ds(cid_of(k) * IB, IB)], so)
        return carry

    lax.fori_loop(0, GROWS // 2, pair, 0)

    drain_store(ra0, so0)
    drain_store(ra1, so1)


def _sc_gather(xi_proj, xj_proj, dst2, src2):
    mesh = plsc.VectorSubcoreMesh(core_axis_name="c", subcore_axis_name="s",
                                  num_cores=NC, num_subcores=NS)
    return pl.kernel(
        _sc_gather_body,
        out_type=jax.ShapeDtypeStruct((E, D), jnp.float32),
        mesh=mesh,
        scratch_types=[
            pltpu.VMEM((GROWS, IB), jnp.int32),
            pltpu.VMEM((GROWS, IB), jnp.int32),
            pltpu.VMEM((IB, D), jnp.float32),
            pltpu.VMEM((IB, D), jnp.float32),
            pltpu.VMEM((IB, D), jnp.float32),
            pltpu.VMEM((IB, D), jnp.float32),
            pltpu.SemaphoreType.DMA,
            pltpu.SemaphoreType.DMA,
            pltpu.SemaphoreType.DMA,
            pltpu.SemaphoreType.DMA,
            pltpu.SemaphoreType.DMA,
            pltpu.SemaphoreType.DMA,
        ],
    )(xi_proj, xj_proj, dst2, src2)


def _sc_scatter_body(chunk_lo, n_chunks, msg_hbm, dst_hbm, out_hbm,
                     idx, rows, acc):
    c = lax.axis_index("c")
    s = lax.axis_index("s")

    def zrow(r, carry):
        for j in range(D // 16):
            rows[r, pl.ds(j * 16, 16)] = jnp.zeros((16,), jnp.float32)
        return carry

    lax.fori_loop(0, ZCH, zrow, 0)

    def zcopy(i, carry):
        cid = s + NS * i

        @pl.when(cid < ZCHUNKS)
        def _():
            pltpu.sync_copy(rows.at[pl.ds(0, ZCH)],
                            acc.at[pl.ds(cid * ZCH, ZCH)])
        return carry

    lax.fori_loop(0, ZITERS, zcopy, 0)
    plsc.subcore_barrier()

    def chunk(i, carry):
        j = s + NS * i
        local = NC * j + c

        @pl.when(local < n_chunks)
        def _():
            irow = (chunk_lo + local) * KROWS
            pltpu.sync_copy(dst_hbm.at[pl.ds(irow, KROWS)], idx)
            pltpu.sync_copy(msg_hbm.at[pl.ds(local * CH, CH)], rows)
            for k in range(KROWS):
                pltpu.sync_copy(rows.at[pl.ds(k * IB, IB)],
                                acc.at[idx.at[k]], add=True)
        return carry

    per_sc = -(-n_chunks // NC)
    lax.fori_loop(0, -(-per_sc // NS), chunk, 0)
    plsc.subcore_barrier()

    def ocopy(i, carry):
        cid = s + NS * i

        @pl.when(cid < ZCHUNKS)
        def _():
            pltpu.sync_copy(acc.at[pl.ds(cid * ZCH, ZCH)],
                            out_hbm.at[c].at[pl.ds(cid * ZCH, ZCH)])
        return carry

    lax.fori_loop(0, ZITERS, ocopy, 0)


def _sc_scatter(msg, dst2, chunk_lo, n_chunks):
    mesh = plsc.VectorSubcoreMesh(core_axis_name="c", subcore_axis_name="s",
                                  num_cores=NC, num_subcores=NS)
    return pl.kernel(
        functools.partial(_sc_scatter_body, chunk_lo, n_chunks),
        out_type=jax.ShapeDtypeStruct((NC, N, D), jnp.float32),
        mesh=mesh,
        scratch_types=[
            pltpu.VMEM((KROWS, IB), jnp.int32),
            pltpu.VMEM((CH, D), jnp.float32),
            pltpu.VMEM_SHARED((N, D), jnp.float32),
        ],
    )(msg, dst2)


def _proj_body(x_ref, wmi_ref, wmj_ref, we2_ref, wme_ref, wg_ref, be2_ref,
               bm1_ref, bg_ref, xi_ref, xj_ref, we2m_ref, we2g_ref,
               bm1p_ref, bgp_ref):
    x = x_ref[...]
    xi_ref[...] = jnp.dot(x, wmi_ref[...], preferred_element_type=jnp.float32)
    xj_ref[...] = jnp.dot(x, wmj_ref[...], preferred_element_type=jnp.float32)
    we2 = we2_ref[...]
    be2 = be2_ref[...]
    wme = wme_ref[...]
    wg = wg_ref[...]
    we2m_ref[...] = jnp.dot(we2, wme, preferred_element_type=jnp.float32)
    we2g_ref[...] = jnp.dot(we2, wg, preferred_element_type=jnp.float32)
    bm1p_ref[...] = jnp.dot(be2, wme, preferred_element_type=jnp.float32) \
        + bm1_ref[...]
    bgp_ref[...] = jnp.dot(be2, wg, preferred_element_type=jnp.float32) \
        + bg_ref[...]


def _node_proj(x, wmi, wmj, we2, wme, wg, be2, bm1, bg):
    return pl.pallas_call(
        _proj_body,
        out_shape=(
            jax.ShapeDtypeStruct((N, D), jnp.float32),
            jax.ShapeDtypeStruct((N, D), jnp.float32),
            jax.ShapeDtypeStruct((D, D), jnp.float32),
            jax.ShapeDtypeStruct((D, D), jnp.float32),
            jax.ShapeDtypeStruct((1, D), jnp.float32),
            jax.ShapeDtypeStruct((1, D), jnp.float32),
        ),
    )(x, wmi, wmj, we2, wme, wg, be2.reshape(1, D), bm1.reshape(1, D),
      bg.reshape(1, D))


def _edge_body(ea_ref, g_ref, we1_ref, be1_ref, we2m_ref, bm1p_ref,
               wm2_ref, bm2_ref, we2g_ref, bgp_ref, out_ref):
    def mm(a, w_ref):
        return jnp.dot(a, w_ref[...], preferred_element_type=jnp.float32)

    ea = ea_ref[...]
    h1 = _silu(mm(ea, we1_ref) + be1_ref[...])
    pre = g_ref[...] + mm(h1, we2m_ref) + bm1p_ref[...]
    msg = mm(_silu(pre), wm2_ref) + bm2_ref[...]
    gate = jax.nn.sigmoid(mm(h1, we2g_ref) + bgp_ref[...])
    out_ref[...] = msg * gate


def _edge_compute(edge_attr, g, we1, be1, we2m, bm1p, wm2, bm2, we2g, bgp,
                  blk_lo, nblk):

    def wfull(shape):
        return pl.BlockSpec(shape, lambda i: (0, 0))

    return pl.pallas_call(
        _edge_body,
        grid=(nblk,),
        in_specs=[
            pl.BlockSpec((EBLK, R), lambda i: (i + blk_lo, 0)),
            pl.BlockSpec((EBLK, D), lambda i: (i + blk_lo, 0)),
            wfull((R, D)),
            wfull((1, D)),
            wfull((D, D)),
            wfull((1, D)),
            wfull((D, D)),
            wfull((1, D)),
            wfull((D, D)),
            wfull((1, D)),
        ],
        out_specs=pl.BlockSpec((EBLK, D), lambda i: (i, 0)),
        out_shape=jax.ShapeDtypeStruct((nblk * EBLK, D), jnp.float32),
    )(edge_attr, g, we1, be1.reshape(1, D), we2m, bm1p, wm2,
      bm2.reshape(1, D), we2g, bgp)


def _update_body(*refs):
    part_refs = refs[:2 * NSLICES]
    (x_ref, wua_ref, wux_ref, bu1_ref, wu2_ref, bu2_ref, g_ref, b_ref,
     out_ref) = refs[2 * NSLICES:]
    x = x_ref[...]
    aggr = part_refs[0][...]
    for pr in part_refs[1:]:
        aggr = aggr + pr[...]
    u1 = _silu(jnp.dot(aggr, wua_ref[...], preferred_element_type=jnp.float32)
               + jnp.dot(x, wux_ref[...], preferred_element_type=jnp.float32)
               + bu1_ref[...])
    h = x + jnp.dot(u1, wu2_ref[...], preferred_element_type=jnp.float32) \
        + bu2_ref[...]
    mu = jnp.mean(h, axis=-1, keepdims=True)
    hc = h - mu
    var = jnp.mean(hc * hc, axis=-1, keepdims=True)
    out_ref[...] = hc * jax.lax.rsqrt(var + 1e-5) * g_ref[...] + b_ref[...]


UBLK = 2000


def _node_update(parts, x, wua, wux, bu1, wu2, bu2, gamma, beta):
    def nspec():
        return pl.BlockSpec((UBLK, D), lambda i: (i, 0))

    def wfull(shape):
        return pl.BlockSpec(shape, lambda i: (0, 0))

    return pl.pallas_call(
        _update_body,
        grid=(N // UBLK,),
        in_specs=[nspec() for _ in parts] + [
            nspec(),
            wfull((D, D)), wfull((D, D)), wfull((1, D)),
            wfull((D, D)), wfull((1, D)), wfull((1, D)), wfull((1, D)),
        ],
        out_specs=nspec(),
        out_shape=jax.ShapeDtypeStruct((N, D), jnp.float32),
    )(*parts, x, wua, wux, bu1.reshape(1, D), wu2, bu2.reshape(1, D),
      gamma.reshape(1, D), beta.reshape(1, D))


def kernel(x, edge_index, edge_attr, We1, be1, We2, be2, Wm1, bm1, Wm2, bm2,
           Wu1, bu1, Wu2, bu2, Wg, bg, ln_gamma, ln_beta):
    src2 = edge_index[0].reshape(E // IB, IB)
    dst2 = edge_index[1].reshape(E // IB, IB)

    wmi = Wm1[:D]
    wmj = Wm1[D:2 * D]
    wme = Wm1[2 * D:]

    xi_proj, xj_proj, we2m, we2g, bm1p, bgp = _node_proj(
        x, wmi, wmj, We2, wme, Wg, be2, bm1, bg)

    g = _sc_gather(xi_proj, xj_proj, dst2, src2)

    nblk_sl = E // EBLK // NSLICES
    chunks_sl = TOTAL_CHUNKS // NSLICES
    parts = []
    for k in range(NSLICES):
        msg_k = _edge_compute(edge_attr, g, We1, be1, we2m, bm1p, Wm2, bm2,
                              we2g, bgp, k * nblk_sl, nblk_sl)
        pk = _sc_scatter(msg_k, dst2, k * chunks_sl, chunks_sl)
        parts.extend([pk[0], pk[1]])

    wua = Wu1[:D]
    wux = Wu1[D:]
    return _node_update(parts, x, wua, wux, bu1, Wu2, bu2, ln_gamma, ln_beta)


# speedup vs baseline: 1.1212x; 1.1212x over previous
import functools
import jax
import jax.numpy as jnp
from jax import lax
from jax.experimental import pallas as pl
from jax.experimental.pallas import tpu as pltpu
from jax.experimental.pallas import tpu_sc as plsc

N = 10000
E = 320000
D = 128
R = 32

NC = 2
NS = 16
NW = NC * NS

IB = 128
KROWS = 2
CH = IB * KROWS
TOTAL_CHUNKS = E // CH
GITERS = -(-TOTAL_CHUNKS // NW)
SITERS = -(-(TOTAL_CHUNKS // NC) // NS)
ZCH = 80
ZCHUNKS = N // ZCH
ZITERS = -(-ZCHUNKS // NS)

EBLK = 2000
NSLICES = 5


def _silu(v):
    return v * jax.nn.sigmoid(v)


GROWS = 80
GLAST = (E // IB) - (NW - 1) * GROWS


def _sc_gather_body(xi_hbm, xj_hbm, dst_hbm, src_hbm, g_hbm,
                    idx_d, idx_s, ra0, rb0, ra1, rb1,
                    sa0, sb0, sa1, sb1, so0, so1):
    wid = lax.axis_index("s") * NC + lax.axis_index("c")
    cnt = jnp.where(wid < NW - 1, GROWS, GLAST)

    @pl.when(wid < NW - 1)
    def _():
        pltpu.sync_copy(dst_hbm.at[pl.ds(wid * GROWS, GROWS)], idx_d)
        pltpu.sync_copy(src_hbm.at[pl.ds(wid * GROWS, GROWS)], idx_s)

    @pl.when(wid == NW - 1)
    def _():
        pltpu.sync_copy(dst_hbm.at[pl.ds((NW - 1) * GROWS, GLAST)],
                        idx_d.at[pl.ds(0, GLAST)])
        pltpu.sync_copy(src_hbm.at[pl.ds((NW - 1) * GROWS, GLAST)],
                        idx_s.at[pl.ds(0, GLAST)])

    def cid_of(k):
        return wid * GROWS + k

    def issue(k, ra, rb, sa, sb):
        pltpu.async_copy(xi_hbm.at[idx_d.at[k]], ra, sa)
        pltpu.async_copy(xj_hbm.at[idx_s.at[k]], rb, sb)

    def drain_gather(ra, rb, sa, sb):
        pltpu.make_async_copy(xi_hbm.at[idx_d.at[0]], ra, sa).wait()
        pltpu.make_async_copy(xj_hbm.at[idx_s.at[0]], rb, sb).wait()

    def drain_store(ra, so):
        pltpu.make_async_copy(ra, g_hbm.at[pl.ds(0, IB)], so).wait()

    issue(0, ra0, rb0, sa0, sb0)

    bufs = ((ra0, rb0, sa0, sb0, so0), (ra1, rb1, sa1, sb1, so1))

    def pair(k2, carry):
        for p in (0, 1):
            k = 2 * k2 + p
            ra, rb, sa, sb, so = bufs[p]
            nra, nrb, nsa, nsb, nso = bufs[1 - p]

            @pl.when((k + 1 < cnt) & (k >= 1))
            def _():
                drain_store(nra, nso)

            @pl.when(k + 1 < cnt)
            def _():
                issue(k + 1, nra, nrb, nsa, nsb)

            @pl.when(k < cnt)
            def _():
                drain_gather(ra, rb, sa, sb)

                def add_row(r, c2):
                    for j in range(D // 16):
                        sl = pl.ds(j * 16, 16)
                        ra[r, sl] = ra[r, sl] + rb[r, sl]
                    return c2

                lax.fori_loop(0, IB, add_row, 0)
                pltpu.async_copy(ra, g_hbm.at[pl.ds(cid_of(k) * IB, IB)], so)
        return carry

    lax.fori_loop(0, GROWS // 2, pair, 0)

    drain_store(ra0, so0)
    drain_store(ra1, so1)


def _sc_gather(xi_proj, xj_proj, dst2, src2):
    mesh = plsc.VectorSubcoreMesh(core_axis_name="c", subcore_axis_name="s",
                                  num_cores=NC, num_subcores=NS)
    return pl.kernel(
        _sc_gather_body,
        out_type=jax.ShapeDtypeStruct((E, D), jnp.float32),
        mesh=mesh,
        scratch_types=[
            pltpu.VMEM((GROWS, IB), jnp.int32),
            pltpu.VMEM((GROWS, IB), jnp.int32),
            pltpu.VMEM((IB, D), jnp.float32),
            pltpu.VMEM((IB, D), jnp.float32),
            pltpu.VMEM((IB, D), jnp.float32),
            pltpu.VMEM((IB, D), jnp.float32),
            pltpu.SemaphoreType.DMA,
            pltpu.SemaphoreType.DMA,
            pltpu.SemaphoreType.DMA,
            pltpu.SemaphoreType.DMA,
            pltpu.SemaphoreType.DMA,
            pltpu.SemaphoreType.DMA,
        ],
    )(xi_proj, xj_proj, dst2, src2)


def _sc_scatter_body(chunk_lo, n_chunks, msg_hbm, dst_hbm, out_hbm,
                     idx0, rows0, idx1, rows1, sl0, sl1, acc):
    c = lax.axis_index("c")
    s = lax.axis_index("s")
    idx = idx0
    rows = rows0

    def zrow(r, carry):
        for j in range(D // 16):
            rows[r, pl.ds(j * 16, 16)] = jnp.zeros((16,), jnp.float32)
        return carry

    lax.fori_loop(0, ZCH, zrow, 0)

    def zcopy(i, carry):
        cid = s + NS * i

        @pl.when(cid < ZCHUNKS)
        def _():
            pltpu.sync_copy(rows.at[pl.ds(0, ZCH)],
                            acc.at[pl.ds(cid * ZCH, ZCH)])
        return carry

    lax.fori_loop(0, ZITERS, zcopy, 0)
    plsc.subcore_barrier()

    def local_of(i):
        return NC * (s + NS * i) + c

    def issue(i, idxb, rowsb, sem):
        local = local_of(i)
        pltpu.async_copy(dst_hbm.at[pl.ds(chunk_lo + local, 1)], idxb, sem)
        pltpu.async_copy(msg_hbm.at[pl.ds(local * IB, IB)], rowsb, sem)

    def drain(idxb, rowsb, sem):
        pltpu.make_async_copy(dst_hbm.at[pl.ds(0, 1)], idxb, sem).wait()
        pltpu.make_async_copy(msg_hbm.at[pl.ds(0, IB)], rowsb, sem).wait()

    @pl.when(local_of(0) < n_chunks)
    def _():
        issue(0, idx0, rows0, sl0)

    bufs = ((idx0, rows0, sl0), (idx1, rows1, sl1))
    per_sc = -(-n_chunks // NC)
    niter = -(-per_sc // NS)

    def chunk(i2, carry):
        for p in (0, 1):
            i = 2 * i2 + p
            idxb, rowsb, sem = bufs[p]
            nidx, nrows, nsem = bufs[1 - p]

            @pl.when(local_of(i + 1) < n_chunks)
            def _():
                issue(i + 1, nidx, nrows, nsem)

            @pl.when(local_of(i) < n_chunks)
            def _():
                drain(idxb, rowsb, sem)
                pltpu.sync_copy(rowsb, acc.at[idxb.at[0]], add=True)
        return carry

    lax.fori_loop(0, (niter + 1) // 2, chunk, 0)
    plsc.subcore_barrier()

    def ocopy(i, carry):
        cid = s + NS * i

        @pl.when(cid < ZCHUNKS)
        def _():
            pltpu.sync_copy(acc.at[pl.ds(cid * ZCH, ZCH)],
                            out_hbm.at[c].at[pl.ds(cid * ZCH, ZCH)])
        return carry

    lax.fori_loop(0, ZITERS, ocopy, 0)


def _sc_scatter(msg, dst2, chunk_lo, n_chunks):
    mesh = plsc.VectorSubcoreMesh(core_axis_name="c", subcore_axis_name="s",
                                  num_cores=NC, num_subcores=NS)
    return pl.kernel(
        functools.partial(_sc_scatter_body, chunk_lo, n_chunks),
        out_type=jax.ShapeDtypeStruct((NC, N, D), jnp.float32),
        mesh=mesh,
        scratch_types=[
            pltpu.VMEM((1, IB), jnp.int32),
            pltpu.VMEM((IB, D), jnp.float32),
            pltpu.VMEM((1, IB), jnp.int32),
            pltpu.VMEM((IB, D), jnp.float32),
            pltpu.SemaphoreType.DMA,
            pltpu.SemaphoreType.DMA,
            pltpu.VMEM_SHARED((N, D), jnp.float32),
        ],
    )(msg, dst2)


def _proj_body(x_ref, wmi_ref, wmj_ref, we2_ref, wme_ref, wg_ref, be2_ref,
               bm1_ref, bg_ref, xi_ref, xj_ref, we2m_ref, we2g_ref,
               bm1p_ref, bgp_ref):
    x = x_ref[...]
    xi_ref[...] = jnp.dot(x, wmi_ref[...], preferred_element_type=jnp.float32)
    xj_ref[...] = jnp.dot(x, wmj_ref[...], preferred_element_type=jnp.float32)
    we2 = we2_ref[...]
    be2 = be2_ref[...]
    wme = wme_ref[...]
    wg = wg_ref[...]
    we2m_ref[...] = jnp.dot(we2, wme, preferred_element_type=jnp.float32)
    we2g_ref[...] = jnp.dot(we2, wg, preferred_element_type=jnp.float32)
    bm1p_ref[...] = jnp.dot(be2, wme, preferred_element_type=jnp.float32) \
        + bm1_ref[...]
    bgp_ref[...] = jnp.dot(be2, wg, preferred_element_type=jnp.float32) \
        + bg_ref[...]


def _node_proj(x, wmi, wmj, we2, wme, wg, be2, bm1, bg):
    return pl.pallas_call(
        _proj_body,
        out_shape=(
            jax.ShapeDtypeStruct((N, D), jnp.float32),
            jax.ShapeDtypeStruct((N, D), jnp.float32),
            jax.ShapeDtypeStruct((D, D), jnp.float32),
            jax.ShapeDtypeStruct((D, D), jnp.float32),
            jax.ShapeDtypeStruct((1, D), jnp.float32),
            jax.ShapeDtypeStruct((1, D), jnp.float32),
        ),
    )(x, wmi, wmj, we2, wme, wg, be2.reshape(1, D), bm1.reshape(1, D),
      bg.reshape(1, D))


def _edge_body(ea_ref, g_ref, we1_ref, be1_ref, we2m_ref, bm1p_ref,
               wm2_ref, bm2_ref, we2g_ref, bgp_ref, out_ref):
    def mm(a, w_ref):
        return jnp.dot(a, w_ref[...], preferred_element_type=jnp.float32)

    ea = ea_ref[...]
    h1 = _silu(mm(ea, we1_ref) + be1_ref[...])
    pre = g_ref[...] + mm(h1, we2m_ref) + bm1p_ref[...]
    msg = mm(_silu(pre), wm2_ref) + bm2_ref[...]
    gate = jax.nn.sigmoid(mm(h1, we2g_ref) + bgp_ref[...])
    out_ref[...] = msg * gate


def _edge_compute(edge_attr, g, we1, be1, we2m, bm1p, wm2, bm2, we2g, bgp,
                  blk_lo, nblk):

    def wfull(shape):
        return pl.BlockSpec(shape, lambda i: (0, 0))

    return pl.pallas_call(
        _edge_body,
        grid=(nblk,),
        in_specs=[
            pl.BlockSpec((EBLK, R), lambda i: (i + blk_lo, 0)),
            pl.BlockSpec((EBLK, D), lambda i: (i + blk_lo, 0)),
            wfull((R, D)),
            wfull((1, D)),
            wfull((D, D)),
            wfull((1, D)),
            wfull((D, D)),
            wfull((1, D)),
            wfull((D, D)),
            wfull((1, D)),
        ],
        out_specs=pl.BlockSpec((EBLK, D), lambda i: (i, 0)),
        out_shape=jax.ShapeDtypeStruct((nblk * EBLK, D), jnp.float32),
    )(edge_attr, g, we1, be1.reshape(1, D), we2m, bm1p, wm2,
      bm2.reshape(1, D), we2g, bgp)


def _update_body(*refs):
    part_refs = refs[:2 * NSLICES]
    (x_ref, wua_ref, wux_ref, bu1_ref, wu2_ref, bu2_ref, g_ref, b_ref,
     out_ref) = refs[2 * NSLICES:]
    x = x_ref[...]
    aggr = part_refs[0][...]
    for pr in part_refs[1:]:
        aggr = aggr + pr[...]
    u1 = _silu(jnp.dot(aggr, wua_ref[...], preferred_element_type=jnp.float32)
               + jnp.dot(x, wux_ref[...], preferred_element_type=jnp.float32)
               + bu1_ref[...])
    h = x + jnp.dot(u1, wu2_ref[...], preferred_element_type=jnp.float32) \
        + bu2_ref[...]
    mu = jnp.mean(h, axis=-1, keepdims=True)
    hc = h - mu
    var = jnp.mean(hc * hc, axis=-1, keepdims=True)
    out_ref[...] = hc * jax.lax.rsqrt(var + 1e-5) * g_ref[...] + b_ref[...]


UBLK = 2000


def _node_update(parts, x, wua, wux, bu1, wu2, bu2, gamma, beta):
    def nspec():
        return pl.BlockSpec((UBLK, D), lambda i: (i, 0))

    def wfull(shape):
        return pl.BlockSpec(shape, lambda i: (0, 0))

    return pl.pallas_call(
        _update_body,
        grid=(N // UBLK,),
        in_specs=[nspec() for _ in parts] + [
            nspec(),
            wfull((D, D)), wfull((D, D)), wfull((1, D)),
            wfull((D, D)), wfull((1, D)), wfull((1, D)), wfull((1, D)),
        ],
        out_specs=nspec(),
        out_shape=jax.ShapeDtypeStruct((N, D), jnp.float32),
    )(*parts, x, wua, wux, bu1.reshape(1, D), wu2, bu2.reshape(1, D),
      gamma.reshape(1, D), beta.reshape(1, D))


def kernel(x, edge_index, edge_attr, We1, be1, We2, be2, Wm1, bm1, Wm2, bm2,
           Wu1, bu1, Wu2, bu2, Wg, bg, ln_gamma, ln_beta):
    src2 = edge_index[0].reshape(E // IB, IB)
    dst2 = edge_index[1].reshape(E // IB, IB)

    wmi = Wm1[:D]
    wmj = Wm1[D:2 * D]
    wme = Wm1[2 * D:]

    xi_proj, xj_proj, we2m, we2g, bm1p, bgp = _node_proj(
        x, wmi, wmj, We2, wme, Wg, be2, bm1, bg)

    g = _sc_gather(xi_proj, xj_proj, dst2, src2)

    nblk_sl = E // EBLK // NSLICES
    chunks_sl = (E // IB) // NSLICES
    parts = []
    for k in range(NSLICES):
        msg_k = _edge_compute(edge_attr, g, We1, be1, we2m, bm1p, Wm2, bm2,
                              we2g, bgp, k * nblk_sl, nblk_sl)
        pk = _sc_scatter(msg_k, dst2, k * chunks_sl, chunks_sl)
        parts.extend([pk[0], pk[1]])

    wua = Wu1[:D]
    wux = Wu1[D:]
    return _node_update(parts, x, wua, wux, bu1, Wu2, bu2, ln_gamma, ln_beta)


# speedup vs baseline: 1.1231x; 1.0017x over previous
import functools
import jax
import jax.numpy as jnp
from jax import lax
from jax.experimental import pallas as pl
from jax.experimental.pallas import tpu as pltpu
from jax.experimental.pallas import tpu_sc as plsc

N = 10000
E = 320000
D = 128
R = 32

NC = 2
NS = 16
NW = NC * NS

IB = 128
KROWS = 2
CH = IB * KROWS
TOTAL_CHUNKS = E // CH
GITERS = -(-TOTAL_CHUNKS // NW)
SITERS = -(-(TOTAL_CHUNKS // NC) // NS)
ZCH = 80
ZCHUNKS = N // ZCH
ZITERS = -(-ZCHUNKS // NS)

EBLK = 2000
NSLICES = 5


def _silu(v):
    return v * jax.nn.sigmoid(v)


GROWS = 80
GLAST = (E // IB) - (NW - 1) * GROWS


def _sc_gather_body(xi_hbm, xj_hbm, dst_hbm, src_hbm, g_hbm,
                    idx_d, idx_s, ra0, rb0, ra1, rb1, ra2, rb2,
                    sa0, sb0, sa1, sb1, sa2, sb2, so0, so1, so2):
    wid = lax.axis_index("s") * NC + lax.axis_index("c")
    cnt = jnp.where(wid < NW - 1, GROWS, GLAST)

    @pl.when(wid < NW - 1)
    def _():
        pltpu.sync_copy(dst_hbm.at[pl.ds(wid * GROWS, GROWS)], idx_d)
        pltpu.sync_copy(src_hbm.at[pl.ds(wid * GROWS, GROWS)], idx_s)

    @pl.when(wid == NW - 1)
    def _():
        pltpu.sync_copy(dst_hbm.at[pl.ds((NW - 1) * GROWS, GLAST)],
                        idx_d.at[pl.ds(0, GLAST)])
        pltpu.sync_copy(src_hbm.at[pl.ds((NW - 1) * GROWS, GLAST)],
                        idx_s.at[pl.ds(0, GLAST)])

    def cid_of(k):
        return wid * GROWS + k

    def issue(k, ra, rb, sa, sb):
        pltpu.async_copy(xi_hbm.at[idx_d.at[k]], ra, sa)
        pltpu.async_copy(xj_hbm.at[idx_s.at[k]], rb, sb)

    def drain_gather(ra, rb, sa, sb):
        pltpu.make_async_copy(xi_hbm.at[idx_d.at[0]], ra, sa).wait()
        pltpu.make_async_copy(xj_hbm.at[idx_s.at[0]], rb, sb).wait()

    def drain_store(ra, so):
        pltpu.make_async_copy(ra, g_hbm.at[pl.ds(0, IB)], so).wait()

    bufs = ((ra0, rb0, sa0, sb0, so0), (ra1, rb1, sa1, sb1, so1),
            (ra2, rb2, sa2, sb2, so2))

    issue(0, ra0, rb0, sa0, sb0)
    issue(1, ra1, rb1, sa1, sb1)

    def trip(k3, carry):
        for p in (0, 1, 2):
            k = 3 * k3 + p
            ra, rb, sa, sb, so = bufs[p]
            nra, nrb, nsa, nsb, nso = bufs[(p + 2) % 3]

            @pl.when((k + 2 < cnt) & (k >= 1))
            def _():
                drain_store(nra, nso)

            @pl.when(k + 2 < cnt)
            def _():
                issue(k + 2, nra, nrb, nsa, nsb)

            @pl.when(k < cnt)
            def _():
                drain_gather(ra, rb, sa, sb)

                def add_row(r, c2):
                    for j in range(D // 16):
                        sl = pl.ds(j * 16, 16)
                        ra[r, sl] = ra[r, sl] + rb[r, sl]
                    return c2

                lax.fori_loop(0, IB, add_row, 0)
                pltpu.async_copy(ra, g_hbm.at[pl.ds(cid_of(k) * IB, IB)], so)
        return carry

    lax.fori_loop(0, (GROWS + 2) // 3, trip, 0)

    drain_store(ra0, so0)
    drain_store(ra1, so1)
    drain_store(ra2, so2)


def _sc_gather(xi_proj, xj_proj, dst2, src2):
    mesh = plsc.VectorSubcoreMesh(core_axis_name="c", subcore_axis_name="s",
                                  num_cores=NC, num_subcores=NS)
    return pl.kernel(
        _sc_gather_body,
        out_type=jax.ShapeDtypeStruct((E, D), jnp.float32),
        mesh=mesh,
        scratch_types=[
            pltpu.VMEM((GROWS, IB), jnp.int32),
            pltpu.VMEM((GROWS, IB), jnp.int32),
            pltpu.VMEM((IB, D), jnp.float32),
            pltpu.VMEM((IB, D), jnp.float32),
            pltpu.VMEM((IB, D), jnp.float32),
            pltpu.VMEM((IB, D), jnp.float32),
            pltpu.VMEM((IB, D), jnp.float32),
            pltpu.VMEM((IB, D), jnp.float32),
            pltpu.SemaphoreType.DMA,
            pltpu.SemaphoreType.DMA,
            pltpu.SemaphoreType.DMA,
            pltpu.SemaphoreType.DMA,
            pltpu.SemaphoreType.DMA,
            pltpu.SemaphoreType.DMA,
            pltpu.SemaphoreType.DMA,
            pltpu.SemaphoreType.DMA,
            pltpu.SemaphoreType.DMA,
        ],
    )(xi_proj, xj_proj, dst2, src2)


def _sc_scatter_body(chunk_lo, n_chunks, msg_hbm, dst_hbm, out_hbm,
                     idx0, rows0, idx1, rows1, sl0, sl1, acc):
    c = lax.axis_index("c")
    s = lax.axis_index("s")
    idx = idx0
    rows = rows0

    def zrow(r, carry):
        for j in range(D // 16):
            rows[r, pl.ds(j * 16, 16)] = jnp.zeros((16,), jnp.float32)
        return carry

    lax.fori_loop(0, ZCH, zrow, 0)

    def zcopy(i, carry):
        cid = s + NS * i

        @pl.when(cid < ZCHUNKS)
        def _():
            pltpu.sync_copy(rows.at[pl.ds(0, ZCH)],
                            acc.at[pl.ds(cid * ZCH, ZCH)])
        return carry

    lax.fori_loop(0, ZITERS, zcopy, 0)
    plsc.subcore_barrier()

    def local_of(i):
        return NC * (s + NS * i) + c

    def issue(i, idxb, rowsb, sem):
        local = local_of(i)
        pltpu.async_copy(dst_hbm.at[pl.ds(chunk_lo + local, 1)], idxb, sem)
        pltpu.async_copy(msg_hbm.at[pl.ds(local * IB, IB)], rowsb, sem)

    def drain(idxb, rowsb, sem):
        pltpu.make_async_copy(dst_hbm.at[pl.ds(0, 1)], idxb, sem).wait()
        pltpu.make_async_copy(msg_hbm.at[pl.ds(0, IB)], rowsb, sem).wait()

    @pl.when(local_of(0) < n_chunks)
    def _():
        issue(0, idx0, rows0, sl0)

    bufs = ((idx0, rows0, sl0), (idx1, rows1, sl1))
    per_sc = -(-n_chunks // NC)
    niter = -(-per_sc // NS)

    def chunk(i2, carry):
        for p in (0, 1):
            i = 2 * i2 + p
            idxb, rowsb, sem = bufs[p]
            nidx, nrows, nsem = bufs[1 - p]

            @pl.when(local_of(i + 1) < n_chunks)
            def _():
                issue(i + 1, nidx, nrows, nsem)

            @pl.when(local_of(i) < n_chunks)
            def _():
                drain(idxb, rowsb, sem)
                pltpu.sync_copy(rowsb, acc.at[idxb.at[0]], add=True)
        return carry

    lax.fori_loop(0, (niter + 1) // 2, chunk, 0)
    plsc.subcore_barrier()

    def ocopy(i, carry):
        cid = s + NS * i

        @pl.when(cid < ZCHUNKS)
        def _():
            pltpu.sync_copy(acc.at[pl.ds(cid * ZCH, ZCH)],
                            out_hbm.at[c].at[pl.ds(cid * ZCH, ZCH)])
        return carry

    lax.fori_loop(0, ZITERS, ocopy, 0)


def _sc_scatter(msg, dst2, chunk_lo, n_chunks):
    mesh = plsc.VectorSubcoreMesh(core_axis_name="c", subcore_axis_name="s",
                                  num_cores=NC, num_subcores=NS)
    return pl.kernel(
        functools.partial(_sc_scatter_body, chunk_lo, n_chunks),
        out_type=jax.ShapeDtypeStruct((NC, N, D), jnp.float32),
        mesh=mesh,
        scratch_types=[
            pltpu.VMEM((1, IB), jnp.int32),
            pltpu.VMEM((IB, D), jnp.float32),
            pltpu.VMEM((1, IB), jnp.int32),
            pltpu.VMEM((IB, D), jnp.float32),
            pltpu.SemaphoreType.DMA,
            pltpu.SemaphoreType.DMA,
            pltpu.VMEM_SHARED((N, D), jnp.float32),
        ],
    )(msg, dst2)


def _proj_body(x_ref, wmi_ref, wmj_ref, we2_ref, wme_ref, wg_ref, be2_ref,
               bm1_ref, bg_ref, xi_ref, xj_ref, we2m_ref, we2g_ref,
               bm1p_ref, bgp_ref):
    x = x_ref[...]
    xi_ref[...] = jnp.dot(x, wmi_ref[...], preferred_element_type=jnp.float32)
    xj_ref[...] = jnp.dot(x, wmj_ref[...], preferred_element_type=jnp.float32)
    we2 = we2_ref[...]
    be2 = be2_ref[...]
    wme = wme_ref[...]
    wg = wg_ref[...]
    we2m_ref[...] = jnp.dot(we2, wme, preferred_element_type=jnp.float32)
    we2g_ref[...] = jnp.dot(we2, wg, preferred_element_type=jnp.float32)
    bm1p_ref[...] = jnp.dot(be2, wme, preferred_element_type=jnp.float32) \
        + bm1_ref[...]
    bgp_ref[...] = jnp.dot(be2, wg, preferred_element_type=jnp.float32) \
        + bg_ref[...]


def _node_proj(x, wmi, wmj, we2, wme, wg, be2, bm1, bg):
    return pl.pallas_call(
        _proj_body,
        out_shape=(
            jax.ShapeDtypeStruct((N, D), jnp.float32),
            jax.ShapeDtypeStruct((N, D), jnp.float32),
            jax.ShapeDtypeStruct((D, D), jnp.float32),
            jax.ShapeDtypeStruct((D, D), jnp.float32),
            jax.ShapeDtypeStruct((1, D), jnp.float32),
            jax.ShapeDtypeStruct((1, D), jnp.float32),
        ),
    )(x, wmi, wmj, we2, wme, wg, be2.reshape(1, D), bm1.reshape(1, D),
      bg.reshape(1, D))


def _edge_body(ea_ref, g_ref, we1_ref, be1_ref, we2m_ref, bm1p_ref,
               wm2_ref, bm2_ref, we2g_ref, bgp_ref, out_ref):
    def mm(a, w_ref):
        return jnp.dot(a, w_ref[...], preferred_element_type=jnp.float32)

    ea = ea_ref[...]
    h1 = _silu(mm(ea, we1_ref) + be1_ref[...])
    pre = g_ref[...] + mm(h1, we2m_ref) + bm1p_ref[...]
    msg = mm(_silu(pre), wm2_ref) + bm2_ref[...]
    gate = jax.nn.sigmoid(mm(h1, we2g_ref) + bgp_ref[...])
    out_ref[...] = msg * gate


def _edge_compute(edge_attr, g, we1, be1, we2m, bm1p, wm2, bm2, we2g, bgp,
                  blk_lo, nblk):

    def wfull(shape):
        return pl.BlockSpec(shape, lambda i: (0, 0))

    return pl.pallas_call(
        _edge_body,
        grid=(nblk,),
        in_specs=[
            pl.BlockSpec((EBLK, R), lambda i: (i + blk_lo, 0)),
            pl.BlockSpec((EBLK, D), lambda i: (i + blk_lo, 0)),
            wfull((R, D)),
            wfull((1, D)),
            wfull((D, D)),
            wfull((1, D)),
            wfull((D, D)),
            wfull((1, D)),
            wfull((D, D)),
            wfull((1, D)),
        ],
        out_specs=pl.BlockSpec((EBLK, D), lambda i: (i, 0)),
        out_shape=jax.ShapeDtypeStruct((nblk * EBLK, D), jnp.float32),
    )(edge_attr, g, we1, be1.reshape(1, D), we2m, bm1p, wm2,
      bm2.reshape(1, D), we2g, bgp)


def _update_body(*refs):
    part_refs = refs[:2 * NSLICES]
    (x_ref, wua_ref, wux_ref, bu1_ref, wu2_ref, bu2_ref, g_ref, b_ref,
     out_ref) = refs[2 * NSLICES:]
    x = x_ref[...]
    aggr = part_refs[0][...]
    for pr in part_refs[1:]:
        aggr = aggr + pr[...]
    u1 = _silu(jnp.dot(aggr, wua_ref[...], preferred_element_type=jnp.float32)
               + jnp.dot(x, wux_ref[...], preferred_element_type=jnp.float32)
               + bu1_ref[...])
    h = x + jnp.dot(u1, wu2_ref[...], preferred_element_type=jnp.float32) \
        + bu2_ref[...]
    mu = jnp.mean(h, axis=-1, keepdims=True)
    hc = h - mu
    var = jnp.mean(hc * hc, axis=-1, keepdims=True)
    out_ref[...] = hc * jax.lax.rsqrt(var + 1e-5) * g_ref[...] + b_ref[...]


UBLK = 2000


def _node_update(parts, x, wua, wux, bu1, wu2, bu2, gamma, beta):
    def nspec():
        return pl.BlockSpec((UBLK, D), lambda i: (i, 0))

    def wfull(shape):
        return pl.BlockSpec(shape, lambda i: (0, 0))

    return pl.pallas_call(
        _update_body,
        grid=(N // UBLK,),
        in_specs=[nspec() for _ in parts] + [
            nspec(),
            wfull((D, D)), wfull((D, D)), wfull((1, D)),
            wfull((D, D)), wfull((1, D)), wfull((1, D)), wfull((1, D)),
        ],
        out_specs=nspec(),
        out_shape=jax.ShapeDtypeStruct((N, D), jnp.float32),
    )(*parts, x, wua, wux, bu1.reshape(1, D), wu2, bu2.reshape(1, D),
      gamma.reshape(1, D), beta.reshape(1, D))


def kernel(x, edge_index, edge_attr, We1, be1, We2, be2, Wm1, bm1, Wm2, bm2,
           Wu1, bu1, Wu2, bu2, Wg, bg, ln_gamma, ln_beta):
    src2 = edge_index[0].reshape(E // IB, IB)
    dst2 = edge_index[1].reshape(E // IB, IB)

    wmi = Wm1[:D]
    wmj = Wm1[D:2 * D]
    wme = Wm1[2 * D:]

    xi_proj, xj_proj, we2m, we2g, bm1p, bgp = _node_proj(
        x, wmi, wmj, We2, wme, Wg, be2, bm1, bg)

    g = _sc_gather(xi_proj, xj_proj, dst2, src2)

    nblk_sl = E // EBLK // NSLICES
    chunks_sl = (E // IB) // NSLICES
    parts = []
    for k in range(NSLICES):
        msg_k = _edge_compute(edge_attr, g, We1, be1, we2m, bm1p, Wm2, bm2,
                              we2g, bgp, k * nblk_sl, nblk_sl)
        pk = _sc_scatter(msg_k, dst2, k * chunks_sl, chunks_sl)
        parts.extend([pk[0], pk[1]])

    wua = Wu1[:D]
    wux = Wu1[D:]
    return _node_update(parts, x, wua, wux, bu1, Wu2, bu2, ln_gamma, ln_beta)
